# baseline (speedup 1.0000x reference)
import functools

import jax
import jax.numpy as jnp
from jax import lax
from jax.experimental import pallas as pl
from jax.experimental.pallas import tpu as pltpu
from jax.experimental.pallas import tpu_sc as plsc

B, T, E, C = 4, 2048, 8, 512
IN_F = 2048
OUT_F = 2048
OE = OUT_F // E
EC = E * C
INV_SQRT2 = 0.7071067811865476

NC, NS = 2, 16
NW = NC * NS
LANES = 16

TT = 256
TTC = 128
TS = 512
TK = 640

TCH = 32
NDC_S = (T - TS) // (2 * TCH)
NHALF = C // 256

ROWS_W = (T - TK) // 8
RCH = 4
NDC_C = ROWS_W // (2 * RCH)


def _xs_body(x_ref, xs16_ref):
    v = jnp.sum(x_ref[0], axis=-1)
    xs16_ref[...] = v[:, None] * jnp.ones((1, LANES), jnp.float32)


def _prep_body(w1_ref, w2_ref, b2_ref, w1s_ref, w2s_ref, bsum_ref):
    e = pl.program_id(0)
    w1s_ref[0, 0, :] = jnp.sum(w1_ref[...], axis=1)
    acc = jnp.zeros((OE,), jnp.float32)
    for k in range(IN_F // OE):
        acc = acc + jnp.sum(w2_ref[:, k * OE:(k + 1) * OE], axis=0)
    w2s_ref[0, 0, :] = acc

    @pl.when(e == 0)
    def _():
        bsum_ref[0, 0, :] = jnp.full((128,), jnp.sum(b2_ref[...]), jnp.float32)


def _s_tc_body(x_ref, dm_ref, s_ref):
    t = pl.program_id(1)

    @pl.when(t == 0)
    def _():
        s_ref[...] = jnp.zeros_like(s_ref)

    xt = jnp.sum(x_ref[0], axis=-1)
    s_ref[0] += jnp.sum(dm_ref[0] * xt[:, None, None], axis=0)


def _g_body(s1_ref, s2_ref, w1s_ref, b1_ref, w2s_ref, bsum_ref, g_ref):
    sv = s1_ref[0, 0, :] + s2_ref[0, 0, :]
    a = sv[:, None] * w1s_ref[0, 0][None, :] + b1_ref[0, 0][None, :]
    h = 0.5 * a * (1.0 + lax.erf(a * INV_SQRT2))
    g_ref[0, 0, :] = jnp.sum(h * w2s_ref[0, 0][None, :], axis=1) + bsum_ref[0, 0, :1]


def _combine_tc_body(cm_ref, g_ref, out_ref):
    out_ref[0, 0, :] = jnp.sum(cm_ref[0] * g_ref[0][None], axis=(1, 2))


def _fin_body(o16_ref, out_ref):
    out_ref[...] = jnp.sum(o16_ref[...], axis=-1)


def _s_sc_body(dm_hbm, xs_hbm, s_hbm, xsb_a, xsb_b, buf_a, buf_b, acc_v,
               sem_a, sem_b):
    cid = lax.axis_index("c")
    sid = lax.axis_index("s")
    wid = sid * NC + cid
    b = wid // E
    e = wid % E

    def start(t0, dm_buf, xs_buf, sem):
        pltpu.async_copy(dm_hbm.at[b, pl.ds(t0, TCH), e, :], dm_buf, sem)
        pltpu.async_copy(xs_hbm.at[pl.ds(b * T + t0, TCH), :], xs_buf, sem)

    def wait(dm_buf, xs_buf, sem):
        pltpu.make_async_copy(
            dm_hbm.at[0, pl.ds(0, TCH), 0, :], dm_buf, sem).wait()
        pltpu.make_async_copy(
            xs_hbm.at[pl.ds(0, TCH), :], xs_buf, sem).wait()

    zv = jnp.zeros((LANES,), jnp.float32)
    for k in range(C // LANES):
        acc_v[pl.ds(k * LANES, LANES)] = zv

    def compute_chunk(dm_buf, xs_buf):
        for h in range(NHALF):
            zeros = tuple(jnp.zeros((LANES,), jnp.float32) for _ in range(16))

            def tb_body(tb, accs, h=h):
                for u in range(8):
                    tl = tb * 8 + u
                    xt = xs_buf[tl, :]
                    accs = tuple(
                        accs[k] + xt * dm_buf[tl, pl.ds(h * 256 + k * LANES, LANES)]
                        for k in range(16)
                    )
                return accs

            accs = lax.fori_loop(0, TCH // 8, tb_body, zeros)
            for k in range(16):
                off = h * 256 + k * LANES
                acc_v[pl.ds(off, LANES)] = acc_v[pl.ds(off, LANES)] + accs[k]

    start(TS, buf_a, xsb_a, sem_a)

    def dc_body(dc, carry):
        t0 = TS + dc * (2 * TCH)
        start(t0 + TCH, buf_b, xsb_b, sem_b)
        wait(buf_a, xsb_a, sem_a)
        compute_chunk(buf_a, xsb_a)

        @pl.when(dc < NDC_S - 1)
        def _(t0=t0):
            start(t0 + 2 * TCH, buf_a, xsb_a, sem_a)

        wait(buf_b, xsb_b, sem_b)
        compute_chunk(buf_b, xsb_b)
        return carry

    lax.fori_loop(0, NDC_S, dc_body, jnp.int32(0))
    pltpu.sync_copy(acc_v, s_hbm.at[b, e, :])


def _combine_sc_body(cm_hbm, g_hbm, o16_hbm, g_v, buf_a, buf_b, out_v,
                     sem_a, sem_b):
    cid = lax.axis_index("c")
    sid = lax.axis_index("s")
    wid = sid * NC + cid
    b = wid // 8
    wsub = wid % 8
    t0w = TK + wsub * ROWS_W
    lrow0 = b * (T - TK) + wsub * ROWS_W
    pltpu.sync_copy(g_hbm.at[b], g_v)

    def compute_chunk(buf, dc, phase):
        accs = tuple(jnp.zeros((LANES,), jnp.float32) for _ in range(RCH))
        for e in range(E):
            def jc_body(jc, accs, e=e):
                loc = tuple(jnp.zeros((LANES,), jnp.float32) for _ in range(RCH))
                for jj in range(4):
                    off = (jc * 4 + jj) * LANES
                    gc = g_v[e, pl.ds(off, LANES)]
                    loc = tuple(
                        loc[r] + gc * buf[r, e, pl.ds(off, LANES)]
                        for r in range(RCH)
                    )
                return tuple(a + l for a, l in zip(accs, loc))

            accs = lax.fori_loop(0, C // (4 * LANES), jc_body, accs)
        r0 = dc * (2 * RCH) + phase * RCH
        for r in range(RCH):
            out_v[pl.ds((r0 + r) * LANES, LANES)] = accs[r]

    pltpu.async_copy(cm_hbm.at[b, pl.ds(t0w, RCH), :, :], buf_a, sem_a)

    def dc_body(dc, carry):
        t0 = t0w + dc * (2 * RCH)
        pltpu.async_copy(cm_hbm.at[b, pl.ds(t0 + RCH, RCH), :, :], buf_b, sem_b)
        pltpu.make_async_copy(
            cm_hbm.at[0, pl.ds(0, RCH), :, :], buf_a, sem_a).wait()
        compute_chunk(buf_a, dc, 0)

        @pl.when(dc < NDC_C - 1)
        def _(t0=t0):
            pltpu.async_copy(
                cm_hbm.at[b, pl.ds(t0 + 2 * RCH, RCH), :, :], buf_a, sem_a)

        pltpu.make_async_copy(
            cm_hbm.at[0, pl.ds(0, RCH), :, :], buf_b, sem_b).wait()
        compute_chunk(buf_b, dc, 1)
        return carry

    lax.fori_loop(0, NDC_C, dc_body, jnp.int32(0))
    pltpu.sync_copy(out_v, o16_hbm.at[pl.ds(lrow0 * LANES, ROWS_W * LANES)])


_SC_MESH = plsc.VectorSubcoreMesh(core_axis_name="c", subcore_axis_name="s")

_s_sc = functools.partial(
    pl.kernel,
    out_type=jax.ShapeDtypeStruct((B, E, C), jnp.float32),
    mesh=_SC_MESH,
    scratch_types=[
        pltpu.VMEM((TCH, LANES), jnp.float32),
        pltpu.VMEM((TCH, LANES), jnp.float32),
        pltpu.VMEM((TCH, C), jnp.float32),
        pltpu.VMEM((TCH, C), jnp.float32),
        pltpu.VMEM((C,), jnp.float32),
        pltpu.SemaphoreType.DMA,
        pltpu.SemaphoreType.DMA,
    ],
)(_s_sc_body)

_combine_sc = functools.partial(
    pl.kernel,
    out_type=jax.ShapeDtypeStruct((B * (T - TK) * LANES,), jnp.float32),
    mesh=_SC_MESH,
    scratch_types=[
        pltpu.VMEM((E, C), jnp.float32),
        pltpu.VMEM((RCH, E, C), jnp.float32),
        pltpu.VMEM((RCH, E, C), jnp.float32),
        pltpu.VMEM((ROWS_W * LANES,), jnp.float32),
        pltpu.SemaphoreType.DMA,
        pltpu.SemaphoreType.DMA,
    ],
)(_combine_sc_body)


def kernel(x, dispatch_mask, combine_array, W1, b1, W2, b2):
    b1r = b1.reshape(E, 1, OE)
    b2r = b2.reshape(1, OUT_F)
    x3 = x.reshape(B * T // 128, 128, IN_F)

    xs16 = pl.pallas_call(
        _xs_body,
        grid=(B * T // 128,),
        in_specs=[pl.BlockSpec((1, 128, IN_F), lambda i: (i, 0, 0))],
        out_specs=pl.BlockSpec((128, LANES), lambda i: (i, 0)),
        out_shape=jax.ShapeDtypeStruct((B * T, LANES), jnp.float32),
    )(x3)

    w1s, w2s, bsum = pl.pallas_call(
        _prep_body,
        grid=(E,),
        in_specs=[
            pl.BlockSpec((OE, IN_F), lambda e: (e, 0)),
            pl.BlockSpec((OE, IN_F), lambda e: (e, 0)),
            pl.BlockSpec((1, OUT_F), lambda e: (0, 0)),
        ],
        out_specs=[
            pl.BlockSpec((1, 1, OE), lambda e: (e, 0, 0)),
            pl.BlockSpec((1, 1, OE), lambda e: (e, 0, 0)),
            pl.BlockSpec((1, 1, 128), lambda e: (0, 0, 0)),
        ],
        out_shape=[
            jax.ShapeDtypeStruct((E, 1, OE), jnp.float32),
            jax.ShapeDtypeStruct((E, 1, OE), jnp.float32),
            jax.ShapeDtypeStruct((1, 1, 128), jnp.float32),
        ],
    )(W1, W2, b2r)

    s_sc = _s_sc(dispatch_mask, xs16)

    s_tc = pl.pallas_call(
        _s_tc_body,
        grid=(B, TS // TT),
        in_specs=[
            pl.BlockSpec((1, TT, IN_F), lambda b, t: (b, t, 0)),
            pl.BlockSpec((1, TT, E, C), lambda b, t: (b, t, 0, 0)),
        ],
        out_specs=pl.BlockSpec((1, E, C), lambda b, t: (b, 0, 0)),
        out_shape=jax.ShapeDtypeStruct((B, E, C), jnp.float32),
    )(x, dispatch_mask)

    g = pl.pallas_call(
        _g_body,
        grid=(B, E),
        in_specs=[
            pl.BlockSpec((1, 1, C), lambda b, e: (b, 0, e)),
            pl.BlockSpec((1, 1, C), lambda b, e: (b, 0, e)),
            pl.BlockSpec((1, 1, OE), lambda b, e: (e, 0, 0)),
            pl.BlockSpec((1, 1, OE), lambda b, e: (e, 0, 0)),
            pl.BlockSpec((1, 1, OE), lambda b, e: (e, 0, 0)),
            pl.BlockSpec((1, 1, 128), lambda b, e: (0, 0, 0)),
        ],
        out_specs=pl.BlockSpec((1, 1, C), lambda b, e: (b, 0, e)),
        out_shape=jax.ShapeDtypeStruct((B, 1, EC), jnp.float32),
    )(s_tc.reshape(B, 1, EC), s_sc.reshape(B, 1, EC), w1s, b1r, w2s, bsum)

    g3 = g.reshape(B, E, C)

    o16 = _combine_sc(combine_array, g3)

    out_tc = pl.pallas_call(
        _combine_tc_body,
        grid=(B, TK // TTC),
        in_specs=[
            pl.BlockSpec((1, TTC, E, C), lambda b, t: (b, t, 0, 0)),
            pl.BlockSpec((1, E, C), lambda b, t: (b, 0, 0)),
        ],
        out_specs=pl.BlockSpec((1, 1, TTC), lambda b, t: (b * (TK // TTC) + t, 0, 0)),
        out_shape=jax.ShapeDtypeStruct((B * (TK // TTC), 1, TTC), jnp.float32),
    )(combine_array, g3)

    nsc = B * (T - TK)
    out_sc = pl.pallas_call(
        _fin_body,
        grid=(1,),
        in_specs=[pl.BlockSpec((nsc // 128, 128, LANES), lambda i: (0, 0, 0))],
        out_specs=pl.BlockSpec((nsc // 128, 128), lambda i: (0, 0)),
        out_shape=jax.ShapeDtypeStruct((nsc // 128, 128), jnp.float32),
    )(o16.reshape(nsc // 128, 128, LANES))

    return jnp.concatenate(
        [out_tc.reshape(B, TK), out_sc.reshape(B, T - TK)], axis=1)


# speedup vs baseline: 1.0371x; 1.0371x over previous
import functools

import jax
import jax.numpy as jnp
from jax import lax
from jax.experimental import pallas as pl
from jax.experimental.pallas import tpu as pltpu
from jax.experimental.pallas import tpu_sc as plsc

B, T, E, C = 4, 2048, 8, 512
IN_F = 2048
OUT_F = 2048
OE = OUT_F // E
EC = E * C
INV_SQRT2 = 0.7071067811865476

NC, NS = 2, 16
NW = NC * NS
LANES = 16

TT = 256
TTC = 128
TS = 768
TK = 768

TCH = 32
NDC_S = (T - TS) // (2 * TCH)
NHALF = C // 256

ROWS_W = (T - TK) // 8
RCH = 4
NDC_C = ROWS_W // (2 * RCH)


def _xs_body(x_ref, xs16_ref):
    v = jnp.sum(x_ref[0], axis=-1)
    xs16_ref[...] = v[:, None] * jnp.ones((1, LANES), jnp.float32)


def _prep_body(w1_ref, w2_ref, b2_ref, w1s_ref, w2s_ref, bsum_ref):
    e = pl.program_id(0)
    w1s_ref[0, 0, :] = jnp.sum(w1_ref[...], axis=1)
    acc = jnp.zeros((OE,), jnp.float32)
    for k in range(IN_F // OE):
        acc = acc + jnp.sum(w2_ref[:, k * OE:(k + 1) * OE], axis=0)
    w2s_ref[0, 0, :] = acc

    @pl.when(e == 0)
    def _():
        bsum_ref[0, 0, :] = jnp.full((128,), jnp.sum(b2_ref[...]), jnp.float32)


def _s_tc_body(x_ref, dm_ref, s_ref):
    t = pl.program_id(1)

    @pl.when(t == 0)
    def _():
        s_ref[...] = jnp.zeros_like(s_ref)

    xt = jnp.sum(x_ref[0], axis=-1)
    s_ref[0] += jnp.sum(dm_ref[0] * xt[:, None, None], axis=0)


def _g_body(s1_ref, s2_ref, w1s_ref, b1_ref, w2s_ref, bsum_ref, g_ref):
    sv = s1_ref[0, 0, :] + s2_ref[0, 0, :]
    a = sv[:, None] * w1s_ref[0, 0][None, :] + b1_ref[0, 0][None, :]
    h = 0.5 * a * (1.0 + lax.erf(a * INV_SQRT2))
    g_ref[0, 0, :] = jnp.sum(h * w2s_ref[0, 0][None, :], axis=1) + bsum_ref[0, 0, :1]


def _combine_tc_body(cm_ref, g_ref, out_ref):
    out_ref[0, 0, :] = jnp.sum(cm_ref[0] * g_ref[0][None], axis=(1, 2))


def _fin_body(o16_ref, out_ref):
    out_ref[...] = jnp.sum(o16_ref[...], axis=-1)


def _s_sc_body(dm_hbm, xs_hbm, s_hbm, xsb_a, xsb_b, buf_a, buf_b, acc_v,
               sem_a, sem_b):
    cid = lax.axis_index("c")
    sid = lax.axis_index("s")
    wid = sid * NC + cid
    b = wid // E
    e = wid % E

    def start(t0, dm_buf, xs_buf, sem):
        pltpu.async_copy(dm_hbm.at[b, pl.ds(t0, TCH), e, :], dm_buf, sem)
        pltpu.async_copy(xs_hbm.at[pl.ds(b * T + t0, TCH), :], xs_buf, sem)

    def wait(dm_buf, xs_buf, sem):
        pltpu.make_async_copy(
            dm_hbm.at[0, pl.ds(0, TCH), 0, :], dm_buf, sem).wait()
        pltpu.make_async_copy(
            xs_hbm.at[pl.ds(0, TCH), :], xs_buf, sem).wait()

    zv = jnp.zeros((LANES,), jnp.float32)
    for k in range(C // LANES):
        acc_v[pl.ds(k * LANES, LANES)] = zv

    def compute_chunk(dm_buf, xs_buf):
        for h in range(NHALF):
            zeros = tuple(jnp.zeros((LANES,), jnp.float32) for _ in range(16))

            def tb_body(tb, accs, h=h):
                for u in range(8):
                    tl = tb * 8 + u
                    xt = xs_buf[tl, :]
                    accs = tuple(
                        accs[k] + xt * dm_buf[tl, pl.ds(h * 256 + k * LANES, LANES)]
                        for k in range(16)
                    )
                return accs

            accs = lax.fori_loop(0, TCH // 8, tb_body, zeros)
            for k in range(16):
                off = h * 256 + k * LANES
                acc_v[pl.ds(off, LANES)] = acc_v[pl.ds(off, LANES)] + accs[k]

    start(TS, buf_a, xsb_a, sem_a)

    def dc_body(dc, carry):
        t0 = TS + dc * (2 * TCH)
        start(t0 + TCH, buf_b, xsb_b, sem_b)
        wait(buf_a, xsb_a, sem_a)
        compute_chunk(buf_a, xsb_a)

        @pl.when(dc < NDC_S - 1)
        def _(t0=t0):
            start(t0 + 2 * TCH, buf_a, xsb_a, sem_a)

        wait(buf_b, xsb_b, sem_b)
        compute_chunk(buf_b, xsb_b)
        return carry

    lax.fori_loop(0, NDC_S, dc_body, jnp.int32(0))
    pltpu.sync_copy(acc_v, s_hbm.at[b, e, :])


def _combine_sc_body(cm_hbm, g_hbm, o16_hbm, g_v, buf_a, buf_b, out_v,
                     sem_a, sem_b):
    cid = lax.axis_index("c")
    sid = lax.axis_index("s")
    wid = sid * NC + cid
    b = wid // 8
    wsub = wid % 8
    t0w = TK + wsub * ROWS_W
    lrow0 = b * (T - TK) + wsub * ROWS_W
    pltpu.sync_copy(g_hbm.at[b], g_v)

    def compute_chunk(buf, dc, phase):
        accs = tuple(jnp.zeros((LANES,), jnp.float32) for _ in range(RCH))
        for e in range(E):
            def jc_body(jc, accs, e=e):
                loc = tuple(jnp.zeros((LANES,), jnp.float32) for _ in range(RCH))
                for jj in range(4):
                    off = (jc * 4 + jj) * LANES
                    gc = g_v[e, pl.ds(off, LANES)]
                    loc = tuple(
                        loc[r] + gc * buf[r, e, pl.ds(off, LANES)]
                        for r in range(RCH)
                    )
                return tuple(a + l for a, l in zip(accs, loc))

            accs = lax.fori_loop(0, C // (4 * LANES), jc_body, accs)
        r0 = dc * (2 * RCH) + phase * RCH
        for r in range(RCH):
            out_v[pl.ds((r0 + r) * LANES, LANES)] = accs[r]

    pltpu.async_copy(cm_hbm.at[b, pl.ds(t0w, RCH), :, :], buf_a, sem_a)

    def dc_body(dc, carry):
        t0 = t0w + dc * (2 * RCH)
        pltpu.async_copy(cm_hbm.at[b, pl.ds(t0 + RCH, RCH), :, :], buf_b, sem_b)
        pltpu.make_async_copy(
            cm_hbm.at[0, pl.ds(0, RCH), :, :], buf_a, sem_a).wait()
        compute_chunk(buf_a, dc, 0)

        @pl.when(dc < NDC_C - 1)
        def _(t0=t0):
            pltpu.async_copy(
                cm_hbm.at[b, pl.ds(t0 + 2 * RCH, RCH), :, :], buf_a, sem_a)

        pltpu.make_async_copy(
            cm_hbm.at[0, pl.ds(0, RCH), :, :], buf_b, sem_b).wait()
        compute_chunk(buf_b, dc, 1)
        return carry

    lax.fori_loop(0, NDC_C, dc_body, jnp.int32(0))
    pltpu.sync_copy(out_v, o16_hbm.at[pl.ds(lrow0 * LANES, ROWS_W * LANES)])


_SC_MESH = plsc.VectorSubcoreMesh(core_axis_name="c", subcore_axis_name="s")

_s_sc = functools.partial(
    pl.kernel,
    out_type=jax.ShapeDtypeStruct((B, E, C), jnp.float32),
    mesh=_SC_MESH,
    scratch_types=[
        pltpu.VMEM((TCH, LANES), jnp.float32),
        pltpu.VMEM((TCH, LANES), jnp.float32),
        pltpu.VMEM((TCH, C), jnp.float32),
        pltpu.VMEM((TCH, C), jnp.float32),
        pltpu.VMEM((C,), jnp.float32),
        pltpu.SemaphoreType.DMA,
        pltpu.SemaphoreType.DMA,
    ],
)(_s_sc_body)

_combine_sc = functools.partial(
    pl.kernel,
    out_type=jax.ShapeDtypeStruct((B * (T - TK) * LANES,), jnp.float32),
    mesh=_SC_MESH,
    scratch_types=[
        pltpu.VMEM((E, C), jnp.float32),
        pltpu.VMEM((RCH, E, C), jnp.float32),
        pltpu.VMEM((RCH, E, C), jnp.float32),
        pltpu.VMEM((ROWS_W * LANES,), jnp.float32),
        pltpu.SemaphoreType.DMA,
        pltpu.SemaphoreType.DMA,
    ],
)(_combine_sc_body)


def kernel(x, dispatch_mask, combine_array, W1, b1, W2, b2):
    b1r = b1.reshape(E, 1, OE)
    b2r = b2.reshape(1, OUT_F)
    x3 = x.reshape(B * T // 128, 128, IN_F)

    xs16 = pl.pallas_call(
        _xs_body,
        grid=(B * T // 128,),
        in_specs=[pl.BlockSpec((1, 128, IN_F), lambda i: (i, 0, 0))],
        out_specs=pl.BlockSpec((128, LANES), lambda i: (i, 0)),
        out_shape=jax.ShapeDtypeStruct((B * T, LANES), jnp.float32),
    )(x3)

    w1s, w2s, bsum = pl.pallas_call(
        _prep_body,
        grid=(E,),
        in_specs=[
            pl.BlockSpec((OE, IN_F), lambda e: (e, 0)),
            pl.BlockSpec((OE, IN_F), lambda e: (e, 0)),
            pl.BlockSpec((1, OUT_F), lambda e: (0, 0)),
        ],
        out_specs=[
            pl.BlockSpec((1, 1, OE), lambda e: (e, 0, 0)),
            pl.BlockSpec((1, 1, OE), lambda e: (e, 0, 0)),
            pl.BlockSpec((1, 1, 128), lambda e: (0, 0, 0)),
        ],
        out_shape=[
            jax.ShapeDtypeStruct((E, 1, OE), jnp.float32),
            jax.ShapeDtypeStruct((E, 1, OE), jnp.float32),
            jax.ShapeDtypeStruct((1, 1, 128), jnp.float32),
        ],
    )(W1, W2, b2r)

    s_sc = _s_sc(dispatch_mask, xs16)

    s_tc = pl.pallas_call(
        _s_tc_body,
        grid=(B, TS // TT),
        in_specs=[
            pl.BlockSpec((1, TT, IN_F), lambda b, t: (b, t, 0)),
            pl.BlockSpec((1, TT, E, C), lambda b, t: (b, t, 0, 0)),
        ],
        out_specs=pl.BlockSpec((1, E, C), lambda b, t: (b, 0, 0)),
        out_shape=jax.ShapeDtypeStruct((B, E, C), jnp.float32),
    )(x, dispatch_mask)

    g = pl.pallas_call(
        _g_body,
        grid=(B, E),
        in_specs=[
            pl.BlockSpec((1, 1, C), lambda b, e: (b, 0, e)),
            pl.BlockSpec((1, 1, C), lambda b, e: (b, 0, e)),
            pl.BlockSpec((1, 1, OE), lambda b, e: (e, 0, 0)),
            pl.BlockSpec((1, 1, OE), lambda b, e: (e, 0, 0)),
            pl.BlockSpec((1, 1, OE), lambda b, e: (e, 0, 0)),
            pl.BlockSpec((1, 1, 128), lambda b, e: (0, 0, 0)),
        ],
        out_specs=pl.BlockSpec((1, 1, C), lambda b, e: (b, 0, e)),
        out_shape=jax.ShapeDtypeStruct((B, 1, EC), jnp.float32),
    )(s_tc.reshape(B, 1, EC), s_sc.reshape(B, 1, EC), w1s, b1r, w2s, bsum)

    g3 = g.reshape(B, E, C)

    o16 = _combine_sc(combine_array, g3)

    out_tc = pl.pallas_call(
        _combine_tc_body,
        grid=(B, TK // TTC),
        in_specs=[
            pl.BlockSpec((1, TTC, E, C), lambda b, t: (b, t, 0, 0)),
            pl.BlockSpec((1, E, C), lambda b, t: (b, 0, 0)),
        ],
        out_specs=pl.BlockSpec((1, 1, TTC), lambda b, t: (b * (TK // TTC) + t, 0, 0)),
        out_shape=jax.ShapeDtypeStruct((B * (TK // TTC), 1, TTC), jnp.float32),
    )(combine_array, g3)

    nsc = B * (T - TK)
    out_sc = pl.pallas_call(
        _fin_body,
        grid=(1,),
        in_specs=[pl.BlockSpec((nsc // 128, 128, LANES), lambda i: (0, 0, 0))],
        out_specs=pl.BlockSpec((nsc // 128, 128), lambda i: (0, 0)),
        out_shape=jax.ShapeDtypeStruct((nsc // 128, 128), jnp.float32),
    )(o16.reshape(nsc // 128, 128, LANES))

    return jnp.concatenate(
        [out_tc.reshape(B, TK), out_sc.reshape(B, T - TK)], axis=1)
